# baseline (speedup 1.0000x reference)
import functools

import jax
import jax.numpy as jnp
from jax import lax
from jax.experimental import pallas as pl
from jax.experimental.pallas import tpu as pltpu
from jax.experimental.pallas import tpu_sc as plsc

N = 100000
E = 400000
DEG = 8

GRAVITY = 9.81
ICE_DENSITY = 917.0
WATER_DENSITY = 1000.0

NC = 2
NS = 16
NW = NC * NS
L = 16


def _pad_to_workers(n, mult):
    per = -(-n // NW)
    per = -(-per // mult) * mult
    return per * NW, per


N_PAD, CN = _pad_to_workers(N, L)
EW = E // 4
CHB = 1600
NCHB = E // CHB
KB = -(-NCHB // NW)
S = 784
NB = CN // S

_MESH = plsc.VectorSubcoreMesh(core_axis_name="c", subcore_axis_name="s")
_PARAMS = pltpu.CompilerParams(needs_layout_passes=False)


def _wid():
    return lax.axis_index("s") * NC + lax.axis_index("c")


@functools.partial(
    pl.kernel,
    out_type=(
        jax.ShapeDtypeStruct((N_PAD,), jnp.float32),
        jax.ShapeDtypeStruct((N_PAD,), jnp.float32),
        jax.ShapeDtypeStruct((N_PAD,), jnp.float32),
    ),
    mesh=_MESH,
    compiler_params=_PARAMS,
    scratch_types=(
        [pltpu.VMEM((N_PAD,), jnp.float32)]
        + [pltpu.VMEM((CN,), jnp.float32) for _ in range(5)]
        + [pltpu.VMEM((DEG * S,), jnp.int32) for _ in range(2)]
        + [pltpu.SemaphoreType.DMA for _ in range(4)]
    ),
)
def _node_fields(ice_hbm, bed_hbm, adj_hbm, op_hbm, comb_hbm, gate_hbm,
                 bed_tab, ice_v, bed_v, op_v, comb_v, min_v, adj_v0, adj_v1,
                 tab_sem, chunk_sem, adj_sem0, adj_sem1):
    w = _wid()
    base = w * CN
    adjs = [adj_v0, adj_v1]
    adj_sems = [adj_sem0, adj_sem1]

    tab_cpy = pltpu.async_copy(bed_hbm, bed_tab, tab_sem)
    ice_cpy = pltpu.async_copy(ice_hbm.at[pl.ds(base, CN)], ice_v, chunk_sem)
    bed_cpy = pltpu.async_copy(bed_hbm.at[pl.ds(base, CN)], bed_v, chunk_sem)

    def stage(b):
        s = b % 2
        return pltpu.async_copy(
            adj_hbm.at[pl.ds((w * NB + b) * (DEG * S), DEG * S)],
            adjs[s], adj_sems[s])

    adj_pending = {0: stage(0)}
    ice_cpy.wait()
    bed_cpy.wait()

    def nf_body(i, carry):
        sl = pl.ds(i * L, L)
        op = ice_v[sl] * (ICE_DENSITY * GRAVITY)
        op_v[sl] = op
        comb_v[sl] = op + (WATER_DENSITY * GRAVITY) * bed_v[sl]
        return carry

    lax.fori_loop(0, CN // L, nf_body, 0)
    out_op = pltpu.async_copy(op_v, op_hbm.at[pl.ds(base, CN)], chunk_sem)
    out_comb = pltpu.async_copy(comb_v, comb_hbm.at[pl.ds(base, CN)], chunk_sem)

    tab_cpy.wait()
    for b in range(NB):
        s = b % 2
        if b + 1 < NB:
            adj_pending[b + 1] = stage(b + 1)
        adj_pending.pop(b).wait()
        av = adjs[s]

        def min_body(i, carry, av=av, b=b):
            m = plsc.load_gather(bed_tab, [av[pl.ds(i * L, L)]])
            for d in range(1, DEG):
                m = jnp.minimum(
                    m, plsc.load_gather(bed_tab, [av[pl.ds(d * S + i * L, L)]]))
            min_v[pl.ds(b * S + i * L, L)] = m
            return carry

        lax.fori_loop(0, S // L, min_body, 0)

    def gate_body(i, carry):
        sl = pl.ds(i * L, L)
        bed = bed_v[sl]
        ok = (bed < min_v[sl]) & (ice_v[sl] + bed < 1000.0)
        min_v[sl] = jnp.where(ok, 1.0, 0.0)
        return carry

    lax.fori_loop(0, CN // L, gate_body, 0)
    out_gate = pltpu.async_copy(min_v, gate_hbm.at[pl.ds(base, CN)], chunk_sem)
    out_op.wait()
    out_comb.wait()
    out_gate.wait()


@functools.partial(
    pl.kernel,
    out_type=(
        jax.ShapeDtypeStruct((E,), jnp.float32),
        jax.ShapeDtypeStruct((EW,), jnp.int32),
    ),
    mesh=_MESH,
    compiler_params=_PARAMS,
    scratch_types=(
        [pltpu.VMEM((N_PAD,), jnp.float32)]
        + [pltpu.VMEM((CHB,), jnp.int32) for _ in range(3)]
        + [pltpu.VMEM((CHB,), jnp.int32) for _ in range(3)]
        + [pltpu.VMEM((CHB,), jnp.float32) for _ in range(3)]
        + [pltpu.VMEM((CHB,), jnp.int32) for _ in range(3)]
        + [pltpu.VMEM((CHB,), jnp.float32) for _ in range(3)]
        + [pltpu.VMEM((CHB // 4,), jnp.int32) for _ in range(3)]
        + [pltpu.SemaphoreType.DMA for _ in range(7)]
    ),
)
def _edge_gradient(comb_hbm, head_hbm, tail_hbm, len_hbm, stat_hbm,
                   grad_hbm, signw_hbm,
                   tab_v, head_v0, head_v1, head_v2, tail_v0, tail_v1, tail_v2,
                   len_v0, len_v1, len_v2, stat_v0, stat_v1, stat_v2,
                   g_v0, g_v1, g_v2, sw_v0, sw_v1, sw_v2,
                   tab_sem, in_sem0, in_sem1, in_sem2,
                   out_sem0, out_sem1, out_sem2):
    w = _wid()
    heads = [head_v0, head_v1, head_v2]
    tails = [tail_v0, tail_v1, tail_v2]
    lens = [len_v0, len_v1, len_v2]
    stats = [stat_v0, stat_v1, stat_v2]
    gs = [g_v0, g_v1, g_v2]
    sws = [sw_v0, sw_v1, sw_v2]
    in_sems = [in_sem0, in_sem1, in_sem2]
    out_sems = [out_sem0, out_sem1, out_sem2]

    tab_cpy = pltpu.async_copy(comb_hbm, tab_v, tab_sem)

    def cid(k):
        c = w + k * NW
        return jnp.where(c < NCHB, c, w)

    def stage(k):
        s = k % 3
        off = pl.ds(cid(k) * CHB, CHB)
        return [pltpu.async_copy(head_hbm.at[off], heads[s], in_sems[s]),
                pltpu.async_copy(tail_hbm.at[off], tails[s], in_sems[s]),
                pltpu.async_copy(len_hbm.at[off], lens[s], in_sems[s]),
                pltpu.async_copy(stat_hbm.at[off], stats[s], in_sems[s])]

    pending = {0: stage(0), 1: stage(1)}
    out_pending = {}
    tab_cpy.wait()
    for k in range(KB):
        s = k % 3
        if k + 2 < KB:
            pending[k + 2] = stage(k + 2)
        for cp in pending.pop(k):
            cp.wait()
        if k - 3 in out_pending:
            for cp in out_pending.pop(k - 3):
                cp.wait()
        hv, tv, lv, sv, gv, swv = (heads[s], tails[s], lens[s], stats[s],
                                   gs[s], sws[s])

        def body(j, carry):
            word = jnp.zeros((L,), jnp.int32)
            for q in range(4):
                sl = pl.ds(j * 64 + q * L, L)
                gh = plsc.load_gather(tab_v, [hv[sl]])
                gt = plsc.load_gather(tab_v, [tv[sl]])
                g = (gt - gh) / lv[sl]
                g = jnp.where(sv[sl] != 0, 0.0, g)
                gv[sl] = g
                sgn = jnp.where(g > 0.0, 2, jnp.where(g < 0.0, 1, 0))
                word = word | (sgn << (8 * q))
            swv[pl.ds(j * L, L)] = word
            return carry

        lax.fori_loop(0, CHB // 64, body, 0)
        out_pending[k] = [
            pltpu.async_copy(gv, grad_hbm.at[pl.ds(cid(k) * CHB, CHB)],
                             out_sems[s]),
            pltpu.async_copy(swv,
                             signw_hbm.at[pl.ds(cid(k) * (CHB // 4),
                                                CHB // 4)],
                             out_sems[s]),
        ]
    for cps in out_pending.values():
        for cp in cps:
            cp.wait()


@functools.partial(
    pl.kernel,
    out_type=jax.ShapeDtypeStruct((N_PAD,), jnp.float32),
    mesh=_MESH,
    compiler_params=_PARAMS,
    scratch_types=(
        [pltpu.VMEM((EW,), jnp.int32)]
        + [pltpu.VMEM((DEG * S,), jnp.int32) for _ in range(2)]
        + [pltpu.VMEM((CN,), jnp.float32) for _ in range(2)]
        + [pltpu.SemaphoreType.DMA for _ in range(4)]
    ),
)
def _boundaries(signw_hbm, enc_hbm, gate_hbm, out_hbm,
                sign_tab, enc_v0, enc_v1, any_v, gate_v,
                tab_sem, gate_sem, in_sem0, in_sem1):
    w = _wid()
    base = w * CN
    encs = [enc_v0, enc_v1]
    in_sems = [in_sem0, in_sem1]

    tab_cpy = pltpu.async_copy(signw_hbm, sign_tab, tab_sem)
    gate_cpy = pltpu.async_copy(gate_hbm.at[pl.ds(base, CN)], gate_v, gate_sem)

    def stage(b):
        s = b % 2
        return pltpu.async_copy(
            enc_hbm.at[pl.ds((w * NB + b) * (DEG * S), DEG * S)],
            encs[s], in_sems[s])

    pending = {0: stage(0)}
    tab_cpy.wait()
    gate_cpy.wait()
    for b in range(NB):
        s = b % 2
        if b + 1 < NB:
            pending[b + 1] = stage(b + 1)
        pending.pop(b).wait()
        ev = encs[s]

        def any_body(i, carry, ev=ev, b=b):
            acc = jnp.zeros((L,), jnp.int32)
            for d in range(DEG):
                e = ev[pl.ds(d * S + i * L, L)]
                wd = plsc.load_gather(sign_tab, [e & 0x1FFFF])
                acc = acc | ((wd >> (e >> 17)) & 1)
            sl = pl.ds(b * S + i * L, L)
            any_v[sl] = jnp.where(acc != 0, gate_v[sl], 0.0)
            return carry

        lax.fori_loop(0, S // L, any_body, 0)

    pltpu.sync_copy(any_v, out_hbm.at[pl.ds(base, CN)])


def kernel(ice_thickness, bedrock_elevation, length_of_link,
           node_at_link_head, node_at_link_tail, links_at_node,
           link_dirs_at_node, active_adjacent_nodes_at_node,
           status_at_link):
    npad = N_PAD - N
    ice = jnp.pad(ice_thickness, (0, npad))
    bed = jnp.pad(bedrock_elevation, (0, npad))

    def blocked(a2d):
        ap = jnp.pad(a2d, ((0, npad), (0, 0)))
        return ap.reshape(N_PAD // S, S, DEG).transpose(0, 2, 1).reshape(-1)

    v = links_at_node
    dirn = link_dirs_at_node
    widx = ((v >> 6) << 4) | (v & 15)
    samt = jnp.where(dirn == 0, 31,
                     ((v >> 4) & 3) * 8 + (dirn == 1).astype(jnp.int32))
    enc_blk = blocked(widx | (samt << 17))
    adj_blk = blocked(active_adjacent_nodes_at_node)

    overburden, combined, gate = _node_fields(ice, bed, adj_blk)
    grad, signw = _edge_gradient(combined, node_at_link_head,
                                 node_at_link_tail, length_of_link,
                                 status_at_link)
    bnd = _boundaries(signw, enc_blk, gate)

    return (grad, overburden[:N], bnd[:N].astype(jnp.bool_))
